# baseline (speedup 1.0000x reference)
import jax
import jax.numpy as jnp
from jax.experimental import pallas as pl
from jax.experimental.pallas import tpu as pltpu

MODEL_DIM = 768
OUT_DIM = 768
NUM_EXPERTS = 8
GATE_HIDDEN = 64
TOP_K = 2
N_TOKENS = 4096

BT = 512


def _moe_block(x_ref, w1_ref, b1_ref, w2_ref, b2_ref, we_ref, be_ref, out_ref):
    xb = x_ref[...]

    gx = jax.lax.dot_general(
        xb, w1_ref[...], (((1,), (1,)), ((), ())),
        preferred_element_type=jnp.float32)
    gx = jnp.maximum(gx + b1_ref[...], 0.0)
    logits = jax.lax.dot_general(
        gx, w2_ref[...], (((1,), (1,)), ((), ())),
        preferred_element_type=jnp.float32)
    logits = logits + b2_ref[...]

    m = jnp.max(logits, axis=1, keepdims=True)
    ex = jnp.exp(logits - m)
    w = ex / jnp.sum(ex, axis=1, keepdims=True)

    lane = jax.lax.broadcasted_iota(jnp.int32, w.shape, 1)
    m1 = jnp.max(w, axis=1, keepdims=True)
    is1 = w == m1
    i1 = jnp.min(jnp.where(is1, lane, NUM_EXPERTS), axis=1, keepdims=True)
    mask1 = lane == i1
    w_rest = jnp.where(mask1, -1.0, w)
    m2 = jnp.max(w_rest, axis=1, keepdims=True)
    is2 = w_rest == m2
    i2 = jnp.min(jnp.where(is2, lane, NUM_EXPERTS), axis=1, keepdims=True)
    mask2 = lane == i2
    denom = m1 + m2
    gating = (jnp.where(mask1, m1, 0.0) + jnp.where(mask2, m2, 0.0)) / denom

    xb16 = xb.astype(jnp.bfloat16)
    y_all = jax.lax.dot_general(
        xb16, we_ref[...], (((1,), (1,)), ((), ())),
        preferred_element_type=jnp.float32)
    be_flat = be_ref[...].reshape(1, NUM_EXPERTS * OUT_DIM)
    y_all = jnp.maximum(y_all + be_flat, 0.0)

    acc0 = jnp.zeros((xb.shape[0], OUT_DIM), dtype=jnp.float32)
    acc1 = jnp.zeros((xb.shape[0], OUT_DIM), dtype=jnp.float32)
    for e in range(NUM_EXPERTS):
        contrib = gating[:, e][:, None] * y_all[:, e * OUT_DIM:(e + 1) * OUT_DIM]
        if e % 2 == 0:
            acc0 = acc0 + contrib
        else:
            acc1 = acc1 + contrib
    out_ref[...] = acc0 + acc1


@jax.jit
def kernel(x, W1, b1, W2, b2, We, be):
    n = x.shape[0]
    grid = (n // BT,)
    full = lambda shape: pl.BlockSpec(shape, lambda i: (0,) * len(shape))
    return pl.pallas_call(
        _moe_block,
        grid=grid,
        in_specs=[
            pl.BlockSpec((BT, MODEL_DIM), lambda i: (i, 0)),
            full((GATE_HIDDEN, MODEL_DIM)),
            full((1, GATE_HIDDEN)),
            full((NUM_EXPERTS, GATE_HIDDEN)),
            full((1, NUM_EXPERTS)),
            full((NUM_EXPERTS * OUT_DIM, MODEL_DIM)),
            full((NUM_EXPERTS, OUT_DIM)),
        ],
        out_specs=pl.BlockSpec((BT, OUT_DIM), lambda i: (i, 0)),
        out_shape=jax.ShapeDtypeStruct((n, OUT_DIM), jnp.float32),
        compiler_params=pltpu.CompilerParams(
            dimension_semantics=("parallel",)),
    )(x, W1, b1.reshape(1, -1), W2, b2.reshape(1, -1),
      We.astype(jnp.bfloat16).reshape(NUM_EXPERTS * OUT_DIM, MODEL_DIM), be)


# speedup vs baseline: 1.0753x; 1.0753x over previous
import jax
import jax.numpy as jnp
from jax.experimental import pallas as pl
from jax.experimental.pallas import tpu as pltpu

MODEL_DIM = 768
OUT_DIM = 768
NUM_EXPERTS = 8
GATE_HIDDEN = 64
TOP_K = 2
N_TOKENS = 4096

BT = 1024
NCHUNK = 2
EPC = NUM_EXPERTS // NCHUNK


def _moe_block(x_ref, w1_ref, b1_ref, w2_ref, b2_ref, we_ref, be_ref,
               out_ref, we16_ref):
    i = pl.program_id(0)

    @pl.when(i == 0)
    def _cast_weights():
        we16_ref[...] = we_ref[...].astype(jnp.bfloat16)

    @pl.when(i > 0)
    def _compute():
        xb = x_ref[...]

        gx = jax.lax.dot_general(
            xb, w1_ref[...], (((1,), (1,)), ((), ())),
            preferred_element_type=jnp.float32)
        gx = jnp.maximum(gx + b1_ref[...], 0.0)
        logits = jax.lax.dot_general(
            gx, w2_ref[...], (((1,), (1,)), ((), ())),
            preferred_element_type=jnp.float32)
        logits = logits + b2_ref[...]

        m = jnp.max(logits, axis=1, keepdims=True)
        ex = jnp.exp(logits - m)
        w = ex / jnp.sum(ex, axis=1, keepdims=True)

        lane = jax.lax.broadcasted_iota(jnp.int32, w.shape, 1)
        m1 = jnp.max(w, axis=1, keepdims=True)
        is1 = w == m1
        i1 = jnp.min(jnp.where(is1, lane, NUM_EXPERTS), axis=1, keepdims=True)
        mask1 = lane == i1
        w_rest = jnp.where(mask1, -1.0, w)
        m2 = jnp.max(w_rest, axis=1, keepdims=True)
        is2 = w_rest == m2
        i2 = jnp.min(jnp.where(is2, lane, NUM_EXPERTS), axis=1, keepdims=True)
        mask2 = lane == i2
        denom = m1 + m2
        gating = (jnp.where(mask1, m1, 0.0) + jnp.where(mask2, m2, 0.0)) / denom

        xb16 = xb.astype(jnp.bfloat16)
        acc0 = jnp.zeros((BT, OUT_DIM), dtype=jnp.float32)
        acc1 = jnp.zeros((BT, OUT_DIM), dtype=jnp.float32)
        for c in range(NCHUNK):
            wchunk = we16_ref[pl.ds(c * EPC * OUT_DIM, EPC * OUT_DIM), :]
            y = jax.lax.dot_general(
                xb16, wchunk, (((1,), (1,)), ((), ())),
                preferred_element_type=jnp.float32)
            bchunk = be_ref[pl.ds(0, 1),
                            pl.ds(c * EPC * OUT_DIM, EPC * OUT_DIM)]
            y = jnp.maximum(y + bchunk, 0.0)
            for k in range(EPC):
                e = c * EPC + k
                contrib = gating[:, e][:, None] * y[:, k * OUT_DIM:(k + 1) * OUT_DIM]
                if e % 2 == 0:
                    acc0 = acc0 + contrib
                else:
                    acc1 = acc1 + contrib
        out_ref[...] = acc0 + acc1


@jax.jit
def kernel(x, W1, b1, W2, b2, We, be):
    n = x.shape[0]
    grid = (1 + n // BT,)
    full = lambda shape: pl.BlockSpec(shape, lambda i: (0,) * len(shape))
    blk = lambda i: (jnp.maximum(i - 1, 0), 0)
    return pl.pallas_call(
        _moe_block,
        grid=grid,
        in_specs=[
            pl.BlockSpec((BT, MODEL_DIM), blk),
            full((GATE_HIDDEN, MODEL_DIM)),
            full((1, GATE_HIDDEN)),
            full((NUM_EXPERTS, GATE_HIDDEN)),
            full((1, NUM_EXPERTS)),
            full((NUM_EXPERTS * OUT_DIM, MODEL_DIM)),
            full((1, NUM_EXPERTS * OUT_DIM)),
        ],
        out_specs=pl.BlockSpec((BT, OUT_DIM), blk),
        out_shape=jax.ShapeDtypeStruct((n, OUT_DIM), jnp.float32),
        scratch_shapes=[
            pltpu.VMEM((NUM_EXPERTS * OUT_DIM, MODEL_DIM), jnp.bfloat16)],
        compiler_params=pltpu.CompilerParams(
            dimension_semantics=("arbitrary",)),
    )(x, W1, b1.reshape(1, -1), W2, b2.reshape(1, -1),
      We.reshape(NUM_EXPERTS * OUT_DIM, MODEL_DIM),
      be.reshape(1, NUM_EXPERTS * OUT_DIM))


# speedup vs baseline: 1.1508x; 1.0702x over previous
import jax
import jax.numpy as jnp
from jax.experimental import pallas as pl
from jax.experimental.pallas import tpu as pltpu

MODEL_DIM = 768
OUT_DIM = 768
NUM_EXPERTS = 8
GATE_HIDDEN = 64
TOP_K = 2
N_TOKENS = 4096

BT = 512


def _moe_block(x_ref, w1_ref, b1_ref, w2_ref, b2_ref, we_ref, be_ref, out_ref):
    xb = x_ref[...]

    gx = jax.lax.dot_general(
        xb, w1_ref[...], (((1,), (1,)), ((), ())),
        preferred_element_type=jnp.float32)
    gx = jnp.maximum(gx + b1_ref[...], 0.0)
    logits = jax.lax.dot_general(
        gx, w2_ref[...], (((1,), (1,)), ((), ())),
        preferred_element_type=jnp.float32)
    logits = logits + b2_ref[...]

    m = jnp.max(logits, axis=1, keepdims=True)
    ex = jnp.exp(logits - m)
    w = ex / jnp.sum(ex, axis=1, keepdims=True)

    lane = jax.lax.broadcasted_iota(jnp.int32, w.shape, 1)
    m1 = jnp.max(w, axis=1, keepdims=True)
    is1 = w == m1
    i1 = jnp.min(jnp.where(is1, lane, NUM_EXPERTS), axis=1, keepdims=True)
    mask1 = lane == i1
    w_rest = jnp.where(mask1, -1.0, w)
    m2 = jnp.max(w_rest, axis=1, keepdims=True)
    is2 = w_rest == m2
    i2 = jnp.min(jnp.where(is2, lane, NUM_EXPERTS), axis=1, keepdims=True)
    mask2 = lane == i2
    denom = m1 + m2
    gating = (jnp.where(mask1, m1, 0.0) + jnp.where(mask2, m2, 0.0)) / denom

    xb16 = xb.astype(jnp.bfloat16)
    CW = 256
    for j in range(OUT_DIM // CW):
        cols = pl.ds(j * CW, CW)
        acc0 = jnp.zeros((BT, CW), dtype=jnp.float32)
        acc1 = jnp.zeros((BT, CW), dtype=jnp.float32)
        for e in range(NUM_EXPERTS):
            wt = we_ref[e, cols, :].astype(jnp.bfloat16)
            y = jax.lax.dot_general(
                xb16, wt, (((1,), (1,)), ((), ())),
                preferred_element_type=jnp.float32)
            y = jnp.maximum(y + be_ref[pl.ds(e, 1), cols], 0.0)
            contrib = gating[:, e][:, None] * y
            if e % 2 == 0:
                acc0 = acc0 + contrib
            else:
                acc1 = acc1 + contrib
        out_ref[:, cols] = acc0 + acc1


@jax.jit
def kernel(x, W1, b1, W2, b2, We, be):
    n = x.shape[0]
    grid = (n // BT,)
    full = lambda shape: pl.BlockSpec(shape, lambda i: (0,) * len(shape))
    return pl.pallas_call(
        _moe_block,
        grid=grid,
        in_specs=[
            pl.BlockSpec((BT, MODEL_DIM), lambda i: (i, 0)),
            full((GATE_HIDDEN, MODEL_DIM)),
            full((1, GATE_HIDDEN)),
            full((NUM_EXPERTS, GATE_HIDDEN)),
            full((1, NUM_EXPERTS)),
            full((NUM_EXPERTS, OUT_DIM, MODEL_DIM)),
            full((NUM_EXPERTS, OUT_DIM)),
        ],
        out_specs=pl.BlockSpec((BT, OUT_DIM), lambda i: (i, 0)),
        out_shape=jax.ShapeDtypeStruct((n, OUT_DIM), jnp.float32),
        compiler_params=pltpu.CompilerParams(
            dimension_semantics=("parallel",)),
    )(x, W1, b1.reshape(1, -1), W2, b2.reshape(1, -1), We, be)
